# baseline (speedup 1.0000x reference)
import jax
import jax.numpy as jnp
from jax.experimental import pallas as pl
from jax.experimental.pallas import tpu as pltpu

NUM_EXPERTS = 8
HIDDEN = 1024
MT = 1024


def _moe_body(x_ref, wr_ref, br_ref, we_ref, be_ref, out_ref,
              coeff_ref, xb_ref, web_ref):
    xf = x_ref[...]
    scores = jax.lax.dot_general(
        xf, wr_ref[...], (((1,), (1,)), ((), ())),
        preferred_element_type=jnp.float32,
    ) + br_ref[...]
    m = jnp.max(scores, axis=1, keepdims=True)
    p = jnp.exp(scores - m)
    p = p / jnp.sum(p, axis=1, keepdims=True)
    iota = jax.lax.broadcasted_iota(jnp.int32, p.shape, 1)
    m0 = jnp.max(p, axis=1, keepdims=True)
    a0 = jnp.min(jnp.where(p == m0, iota, NUM_EXPERTS), axis=1, keepdims=True)
    p1m = jnp.where(iota == a0, -1.0, p)
    m1 = jnp.max(p1m, axis=1, keepdims=True)
    a1 = jnp.min(jnp.where(p1m == m1, iota, NUM_EXPERTS), axis=1,
                 keepdims=True)
    wsum = m0 + m1
    coeff_ref[...] = (m0 * (iota == a0) + m1 * (iota == a1)) / wsum

    xb_ref[...] = xf.astype(jnp.bfloat16)
    bias = jax.lax.dot_general(
        coeff_ref[...].astype(jnp.bfloat16), be_ref[...],
        (((1,), (0,)), ((), ())),
        preferred_element_type=jnp.float32,
    )
    for e in range(NUM_EXPERTS):
        web_ref[...] = we_ref[e].astype(jnp.bfloat16)
        y = jax.lax.dot_general(
            xb_ref[...], web_ref[...], (((1,), (1,)), ((), ())),
            preferred_element_type=jnp.float32,
        )
        ce = coeff_ref[:, e:e + 1]
        if e == 0:
            out_ref[...] = ce * y + bias
        else:
            out_ref[...] += ce * y


def kernel(x, Wr, br, We, be):
    B, S, H = x.shape
    M = B * S
    xf = x.reshape(M, H)
    br2 = br.reshape(1, NUM_EXPERTS)
    be_bf = be.astype(jnp.bfloat16)
    grid = (M // MT,)
    out = pl.pallas_call(
        _moe_body,
        grid=grid,
        in_specs=[
            pl.BlockSpec((MT, H), lambda t: (t, 0)),
            pl.BlockSpec((NUM_EXPERTS, H), lambda t: (0, 0)),
            pl.BlockSpec((1, NUM_EXPERTS), lambda t: (0, 0)),
            pl.BlockSpec((NUM_EXPERTS, H, H), lambda t: (0, 0, 0)),
            pl.BlockSpec((NUM_EXPERTS, H), lambda t: (0, 0)),
        ],
        out_specs=pl.BlockSpec((MT, H), lambda t: (t, 0)),
        out_shape=jax.ShapeDtypeStruct((M, H), jnp.float32),
        scratch_shapes=[
            pltpu.VMEM((MT, NUM_EXPERTS), jnp.float32),
            pltpu.VMEM((MT, HIDDEN), jnp.bfloat16),
            pltpu.VMEM((HIDDEN, HIDDEN), jnp.bfloat16),
        ],
    )(xf, Wr, br2, We, be_bf)
    return out.reshape(B, S, H)
